# baseline (speedup 1.0000x reference)
import functools

import jax
import jax.numpy as jnp
from jax import lax
from jax.experimental import pallas as pl
from jax.experimental.pallas import tpu as pltpu
from jax.experimental.pallas import tpu_sc as plsc

N = 10000
D = 128
E = 320000
H = 16
C = 10

NC = 2
NS = 16
NW = NC * NS
CHUNK = 256
NCH = 40
EPW = NCH * CHUNK
E_PAD = NW * EPW
VROWS = 10240
ACC_ROWS = 10240
ZROWS = ACC_ROWS // NS
RPT = N // NS


G = 8
ROUNDS = NCH // G


def _sc_segsum_body(vals_hbm, sd_hbm, zeros_hbm, out_hbm,
                    src_v, dst_v, rows_v, acc_sh, vals_sp, *sems):
    c = lax.axis_index("c")
    s = lax.axis_index("s")
    wid = c * NS + s
    pltpu.sync_copy(zeros_hbm.at[pl.ds(s * ZROWS, ZROWS)],
                    acc_sh.at[pl.ds(s * ZROWS, ZROWS)])
    pltpu.sync_copy(vals_hbm.at[pl.ds(s * (VROWS // NS), VROWS // NS)],
                    vals_sp.at[pl.ds(s * (VROWS // NS), VROWS // NS)])
    pltpu.sync_copy(sd_hbm.at[0].at[wid], src_v)
    pltpu.sync_copy(sd_hbm.at[1].at[wid], dst_v)
    plsc.subcore_barrier()

    def start_gather(b, j):
        pltpu.async_copy(vals_sp.at[src_v.at[j]], rows_v.at[b], sems[b])

    def wait_gather(b, j):
        pltpu.make_async_copy(vals_sp.at[src_v.at[j]], rows_v.at[b],
                              sems[b]).wait()

    for b in range(G):
        start_gather(b, b)

    @pl.loop(0, NCH - G, step=G)
    def _(jj):
        for b in range(G):
            wait_gather(b, jj + b)
            pltpu.sync_copy(rows_v.at[b], acc_sh.at[dst_v.at[jj + b]],
                            add=True)
            start_gather(b, jj + G + b)

    for b in range(G):
        jj = NCH - G
        wait_gather(b, jj + b)
        pltpu.sync_copy(rows_v.at[b], acc_sh.at[dst_v.at[jj + b]], add=True)

    plsc.subcore_barrier()
    pltpu.sync_copy(acc_sh.at[pl.ds(s * ZROWS, ZROWS)],
                    out_hbm.at[c].at[pl.ds(s * ZROWS, ZROWS)])


@functools.cache
def _sc_segsum():
    mesh = plsc.VectorSubcoreMesh(core_axis_name="c", subcore_axis_name="s",
                                  num_cores=NC)
    return pl.kernel(
        _sc_segsum_body,
        out_type=jax.ShapeDtypeStruct((NC, ACC_ROWS, 16), jnp.float32),
        mesh=mesh,
        compiler_params=pltpu.CompilerParams(use_tc_tiling_on_sc=False),
        scratch_types=[
            pltpu.VMEM((NCH, CHUNK), jnp.int32),
            pltpu.VMEM((NCH, CHUNK), jnp.int32),
            pltpu.VMEM((G, CHUNK, 16), jnp.float32),
            pltpu.VMEM_SHARED((ACC_ROWS, 16), jnp.float32),
            pltpu.VMEM_SHARED((VROWS, 16), jnp.float32),
        ] + [pltpu.SemaphoreType.DMA] * G,
    )


PN = N // 8
PV = VROWS * 16 // 128


def _dot(a, w):
    return lax.dot_general(a, w, (((1,), (0,)), ((), ())),
                           preferred_element_type=jnp.float32)


def _proj1_body(xb_ref, wrel_ref, wroot_ref, xr_ref, xroot_ref):
    xb = xb_ref[...]
    pr = _dot(xb, wrel_ref[...])
    xr_ref[...] = jnp.concatenate(
        [pr, jnp.zeros((PV - PN, 128), jnp.float32)], axis=0)
    xroot_ref[...] = _dot(xb, wroot_ref[...])


_proj1 = pl.pallas_call(
    _proj1_body,
    out_shape=(jax.ShapeDtypeStruct((PV, 128), jnp.float32),
               jax.ShapeDtypeStruct((PN, 128), jnp.float32)),
)


def _mid_body(parts_ref, xroot_ref, b1_ref, wrel_ref, wroot_ref,
              hr_ref, hroot_ref):
    agg = parts_ref[0, :PN] + parts_ref[1, :PN]
    h = jnp.maximum(agg + xroot_ref[...] + b1_ref[...].reshape(1, 128), 0.0)
    hr = _dot(h, wrel_ref[...])
    hr_ref[...] = jnp.concatenate(
        [hr, jnp.zeros((PV - PN, 128), jnp.float32)], axis=0)
    hroot_ref[...] = _dot(h, wroot_ref[...])


_mid = pl.pallas_call(
    _mid_body,
    out_shape=(jax.ShapeDtypeStruct((PV, 128), jnp.float32),
               jax.ShapeDtypeStruct((PN, 128), jnp.float32)),
)


def _out_body(parts_ref, hroot_ref, b2_ref, o_ref):
    o = (parts_ref[0, :PN] + parts_ref[1, :PN] + hroot_ref[...]
         + b2_ref[...].reshape(1, 128))
    valid = lax.broadcasted_iota(jnp.int32, (1, 16), 1) < C
    outs = []
    for k in range(8):
        ok = o[:, 16 * k:16 * k + 16]
        om = jnp.where(valid, ok, -1e30)
        m = jnp.max(om, axis=1, keepdims=True)
        lse = m + jnp.log(jnp.sum(jnp.exp(om - m), axis=1, keepdims=True))
        outs.append(ok - lse)
    o_ref[...] = jnp.concatenate(outs, axis=1)


_outk = pl.pallas_call(
    _out_body,
    out_shape=jax.ShapeDtypeStruct((PN, 128), jnp.float32),
)


def kernel(x, edge_index, W1_rel, b1, W1_root, W2_rel, b2, W2_root):
    ppw = EPW - E // NW
    pad = jnp.broadcast_to(N + jnp.arange(ppw, dtype=jnp.int32), (2, NW, ppw))
    sd = jnp.concatenate([edge_index.reshape(2, NW, E // NW), pad], axis=2)
    sd = sd.reshape(2, NW, NCH, CHUNK)
    zrows = jnp.zeros((ACC_ROWS, 16), jnp.float32)

    eye8 = jnp.eye(8, dtype=jnp.float32)
    w1rel = jnp.kron(eye8, W1_rel.T)
    w1root = jnp.kron(eye8, W1_root.T)
    w2p = jnp.pad(W2_rel, ((0, H - C), (0, 0)))
    w2rel = jnp.kron(eye8, w2p.T)
    w2rootp = jnp.pad(W2_root, ((0, H - C), (0, 0)))
    w2root = jnp.kron(eye8, w2rootp.T)
    b1t = jnp.tile(b1, 8)
    b2t = jnp.tile(jnp.pad(b2, (0, H - C)), 8)
    xb = x.reshape(PN, 8 * D)

    segsum = _sc_segsum()
    xrp, xrootp = _proj1(xb, w1rel, w1root)
    parts1 = segsum(xrp.reshape(VROWS, 16), sd, zrows)
    hrp, hrootp = _mid(parts1.reshape(NC, PV, 128), xrootp, b1t, w2rel, w2root)
    parts2 = segsum(hrp.reshape(VROWS, 16), sd, zrows)
    outp = _outk(parts2.reshape(NC, PV, 128), hrootp, b2t)
    return outp.reshape(N, 16)[:, :C]


# speedup vs baseline: 1.0345x; 1.0345x over previous
import functools

import jax
import jax.numpy as jnp
from jax import lax
from jax.experimental import pallas as pl
from jax.experimental.pallas import tpu as pltpu
from jax.experimental.pallas import tpu_sc as plsc

N = 10000
D = 128
E = 320000
H = 16
C = 10

NC = 2
NS = 16
NW = NC * NS
CHUNK = 128
NCH = 80
EPW = NCH * CHUNK
E_PAD = NW * EPW
VROWS = 10240
ACC_ROWS = 10240
ZROWS = ACC_ROWS // NS
RPT = N // NS


G = 8
ROUNDS = NCH // G


def _sc_segsum_body(vals_hbm, sd_hbm, zeros_hbm, out_hbm,
                    src_v, dst_v, rows_v, acc_sh, *sems):
    c = lax.axis_index("c")
    s = lax.axis_index("s")
    wid = c * NS + s
    pltpu.sync_copy(zeros_hbm.at[pl.ds(s * ZROWS, ZROWS)],
                    acc_sh.at[pl.ds(s * ZROWS, ZROWS)])
    pltpu.sync_copy(sd_hbm.at[0].at[wid], src_v)
    pltpu.sync_copy(sd_hbm.at[1].at[wid], dst_v)
    plsc.subcore_barrier()

    def start_gather(b, j):
        pltpu.async_copy(vals_hbm.at[src_v.at[j]], rows_v.at[b], sems[b])

    def wait_gather(b, j):
        pltpu.make_async_copy(vals_hbm.at[src_v.at[j]], rows_v.at[b],
                              sems[b]).wait()

    for b in range(G):
        start_gather(b, b)

    @pl.loop(0, NCH - G, step=G)
    def _(jj):
        for b in range(G):
            wait_gather(b, jj + b)
            pltpu.sync_copy(rows_v.at[b], acc_sh.at[dst_v.at[jj + b]],
                            add=True)
            start_gather(b, jj + G + b)

    for b in range(G):
        jj = NCH - G
        wait_gather(b, jj + b)
        pltpu.sync_copy(rows_v.at[b], acc_sh.at[dst_v.at[jj + b]], add=True)

    plsc.subcore_barrier()
    pltpu.sync_copy(acc_sh.at[pl.ds(s * ZROWS, ZROWS)],
                    out_hbm.at[c].at[pl.ds(s * ZROWS, ZROWS)])


@functools.cache
def _sc_segsum():
    mesh = plsc.VectorSubcoreMesh(core_axis_name="c", subcore_axis_name="s",
                                  num_cores=NC)
    return pl.kernel(
        _sc_segsum_body,
        out_type=jax.ShapeDtypeStruct((NC, ACC_ROWS, 16), jnp.float32),
        mesh=mesh,
        compiler_params=pltpu.CompilerParams(use_tc_tiling_on_sc=False),
        scratch_types=[
            pltpu.VMEM((NCH, CHUNK), jnp.int32),
            pltpu.VMEM((NCH, CHUNK), jnp.int32),
            pltpu.VMEM((G, CHUNK, 16), jnp.float32),
            pltpu.VMEM_SHARED((ACC_ROWS, 16), jnp.float32),
        ] + [pltpu.SemaphoreType.DMA] * G,
    )


PN = N // 8
PV = VROWS * 16 // 128


def _dot(a, w):
    return lax.dot_general(a, w, (((1,), (0,)), ((), ())),
                           preferred_element_type=jnp.float32)


def _proj1_body(xb_ref, wrel_ref, wroot_ref, xr_ref, xroot_ref):
    xb = xb_ref[...]
    pr = _dot(xb, wrel_ref[...])
    xr_ref[...] = jnp.concatenate(
        [pr, jnp.zeros((PV - PN, 128), jnp.float32)], axis=0)
    xroot_ref[...] = _dot(xb, wroot_ref[...])


_proj1 = pl.pallas_call(
    _proj1_body,
    out_shape=(jax.ShapeDtypeStruct((PV, 128), jnp.float32),
               jax.ShapeDtypeStruct((PN, 128), jnp.float32)),
)


def _mid_body(parts_ref, xroot_ref, b1_ref, wrel_ref, wroot_ref,
              hr_ref, hroot_ref):
    agg = parts_ref[0, :PN] + parts_ref[1, :PN]
    h = jnp.maximum(agg + xroot_ref[...] + b1_ref[...].reshape(1, 128), 0.0)
    hr = _dot(h, wrel_ref[...])
    hr_ref[...] = jnp.concatenate(
        [hr, jnp.zeros((PV - PN, 128), jnp.float32)], axis=0)
    hroot_ref[...] = _dot(h, wroot_ref[...])


_mid = pl.pallas_call(
    _mid_body,
    out_shape=(jax.ShapeDtypeStruct((PV, 128), jnp.float32),
               jax.ShapeDtypeStruct((PN, 128), jnp.float32)),
)


def _out_body(parts_ref, hroot_ref, b2_ref, o_ref):
    o = (parts_ref[0, :PN] + parts_ref[1, :PN] + hroot_ref[...]
         + b2_ref[...].reshape(1, 128))
    valid = lax.broadcasted_iota(jnp.int32, (1, 16), 1) < C
    outs = []
    for k in range(8):
        ok = o[:, 16 * k:16 * k + 16]
        om = jnp.where(valid, ok, -1e30)
        m = jnp.max(om, axis=1, keepdims=True)
        lse = m + jnp.log(jnp.sum(jnp.exp(om - m), axis=1, keepdims=True))
        outs.append(ok - lse)
    o_ref[...] = jnp.concatenate(outs, axis=1)


_outk = pl.pallas_call(
    _out_body,
    out_shape=jax.ShapeDtypeStruct((PN, 128), jnp.float32),
)


def kernel(x, edge_index, W1_rel, b1, W1_root, W2_rel, b2, W2_root):
    ppw = EPW - E // NW
    pad = jnp.broadcast_to(N + jnp.arange(ppw, dtype=jnp.int32), (2, NW, ppw))
    sd = jnp.concatenate([edge_index.reshape(2, NW, E // NW), pad], axis=2)
    sd = sd.reshape(2, NW, NCH, CHUNK)
    zrows = jnp.zeros((ACC_ROWS, 16), jnp.float32)

    eye8 = jnp.eye(8, dtype=jnp.float32)
    w1rel = jnp.kron(eye8, W1_rel.T)
    w1root = jnp.kron(eye8, W1_root.T)
    w2p = jnp.pad(W2_rel, ((0, H - C), (0, 0)))
    w2rel = jnp.kron(eye8, w2p.T)
    w2rootp = jnp.pad(W2_root, ((0, H - C), (0, 0)))
    w2root = jnp.kron(eye8, w2rootp.T)
    b1t = jnp.tile(b1, 8)
    b2t = jnp.tile(jnp.pad(b2, (0, H - C)), 8)
    xb = x.reshape(PN, 8 * D)

    segsum = _sc_segsum()
    xrp, xrootp = _proj1(xb, w1rel, w1root)
    parts1 = segsum(xrp.reshape(VROWS, 16), sd, zrows)
    hrp, hrootp = _mid(parts1.reshape(NC, PV, 128), xrootp, b1t, w2rel, w2root)
    parts2 = segsum(hrp.reshape(VROWS, 16), sd, zrows)
    outp = _outk(parts2.reshape(NC, PV, 128), hrootp, b2t)
    return outp.reshape(N, 16)[:, :C]


# speedup vs baseline: 1.1620x; 1.1233x over previous
import functools

import jax
import jax.numpy as jnp
from jax import lax
from jax.experimental import pallas as pl
from jax.experimental.pallas import tpu as pltpu
from jax.experimental.pallas import tpu_sc as plsc

N = 10000
D = 128
E = 320000
H = 16
C = 10

NC = 2
NS = 16
NW = NC * NS
CHUNK = 128
NCH = 80
EPW = NCH * CHUNK
E_PAD = NW * EPW
VROWS = 10240
ACC_ROWS = 10240
ZROWS = ACC_ROWS // NS
RPT = N // NS


G = 8
ROUNDS = NCH // G


def _sc_segsum_body(vals_hbm, sd_hbm, zeros_hbm, out_hbm,
                    src_v, dst_v, rows_v, acc_sh, *sems):
    c = lax.axis_index("c")
    s = lax.axis_index("s")
    wid = c * NS + s
    pltpu.sync_copy(zeros_hbm.at[pl.ds(s * ZROWS, ZROWS)],
                    acc_sh.at[pl.ds(s * ZROWS, ZROWS)])
    pltpu.sync_copy(sd_hbm.at[0].at[wid], src_v)
    pltpu.sync_copy(sd_hbm.at[1].at[wid], dst_v)
    plsc.subcore_barrier()

    def start_gather(b, j):
        pltpu.async_copy(vals_hbm.at[src_v.at[j]], rows_v.at[b], sems[b])

    def wait_gather(b, j):
        pltpu.make_async_copy(vals_hbm.at[src_v.at[j]], rows_v.at[b],
                              sems[b]).wait()

    for b in range(G):
        start_gather(b, b)

    @pl.loop(0, NCH - G, step=G)
    def _(jj):
        for b in range(G):
            wait_gather(b, jj + b)
            pltpu.sync_copy(rows_v.at[b], acc_sh.at[dst_v.at[jj + b]],
                            add=True)
            start_gather(b, jj + G + b)

    for b in range(G):
        jj = NCH - G
        wait_gather(b, jj + b)
        pltpu.sync_copy(rows_v.at[b], acc_sh.at[dst_v.at[jj + b]], add=True)

    plsc.subcore_barrier()
    pltpu.sync_copy(acc_sh.at[pl.ds(s * ZROWS, ZROWS)],
                    out_hbm.at[c].at[pl.ds(s * ZROWS, ZROWS)])


@functools.cache
def _sc_segsum():
    mesh = plsc.VectorSubcoreMesh(core_axis_name="c", subcore_axis_name="s",
                                  num_cores=NC)
    return pl.kernel(
        _sc_segsum_body,
        out_type=jax.ShapeDtypeStruct((NC, ACC_ROWS, 16), jnp.float32),
        mesh=mesh,
        compiler_params=pltpu.CompilerParams(use_tc_tiling_on_sc=False),
        scratch_types=[
            pltpu.VMEM((NCH, CHUNK), jnp.int32),
            pltpu.VMEM((NCH, CHUNK), jnp.int32),
            pltpu.VMEM((G, CHUNK, 16), jnp.float32),
            pltpu.VMEM_SHARED((ACC_ROWS, 16), jnp.float32),
        ] + [pltpu.SemaphoreType.DMA] * G,
    )


PN = N // 8
PV = VROWS * 16 // 128


def _dot(a, w):
    return lax.dot_general(a, w, (((1,), (0,)), ((), ())),
                           preferred_element_type=jnp.float32)


def _dott(a, w):
    return lax.dot_general(a, w, (((1,), (1,)), ((), ())),
                           preferred_element_type=jnp.float32)


def _proj1_body(x_ref, wrel_ref, wroot_ref, xr_ref, xroot_ref):
    pr = jnp.concatenate(
        [_dott(x_ref[k::8, :], wrel_ref[...]) for k in range(8)], axis=1)
    xr_ref[...] = jnp.concatenate(
        [pr, jnp.zeros((PV - PN, 128), jnp.float32)], axis=0)
    xroot_ref[...] = jnp.concatenate(
        [_dott(x_ref[k::8, :], wroot_ref[...]) for k in range(8)], axis=1)


_proj1 = pl.pallas_call(
    _proj1_body,
    out_shape=(jax.ShapeDtypeStruct((PV, 128), jnp.float32),
               jax.ShapeDtypeStruct((PN, 128), jnp.float32)),
)


def _mid_body(parts_ref, xroot_ref, b1_ref, wrel_ref, wroot_ref,
              hr_ref, hroot_ref):
    agg = parts_ref[0, :PN] + parts_ref[1, :PN]
    h = jnp.maximum(agg + xroot_ref[...] + b1_ref[...].reshape(1, 128), 0.0)
    hr = _dot(h, wrel_ref[...])
    hr_ref[...] = jnp.concatenate(
        [hr, jnp.zeros((PV - PN, 128), jnp.float32)], axis=0)
    hroot_ref[...] = _dot(h, wroot_ref[...])


_mid = pl.pallas_call(
    _mid_body,
    out_shape=(jax.ShapeDtypeStruct((PV, 128), jnp.float32),
               jax.ShapeDtypeStruct((PN, 128), jnp.float32)),
)


def _out_body(parts_ref, hroot_ref, b2_ref, o_ref):
    o = (parts_ref[0, :PN] + parts_ref[1, :PN] + hroot_ref[...]
         + b2_ref[...].reshape(1, 128))
    valid = lax.broadcasted_iota(jnp.int32, (1, 16), 1) < C
    for k in range(8):
        ok = o[:, 16 * k:16 * k + 16]
        om = jnp.where(valid, ok, -1e30)
        m = jnp.max(om, axis=1, keepdims=True)
        lse = m + jnp.log(jnp.sum(jnp.exp(om - m), axis=1, keepdims=True))
        o_ref[k::8, :] = (ok - lse)[:, :C]


_outk = pl.pallas_call(
    _out_body,
    out_shape=jax.ShapeDtypeStruct((N, C), jnp.float32),
)


def kernel(x, edge_index, W1_rel, b1, W1_root, W2_rel, b2, W2_root):
    ppw = EPW - E // NW
    pad = jnp.broadcast_to(N + jnp.arange(ppw, dtype=jnp.int32), (2, NW, ppw))
    sd = jnp.concatenate([edge_index.reshape(2, NW, E // NW), pad], axis=2)
    sd = sd.reshape(2, NW, NCH, CHUNK)
    zrows = jnp.zeros((ACC_ROWS, 16), jnp.float32)

    eye8 = jnp.eye(8, dtype=jnp.float32)
    w2p = jnp.pad(W2_rel, ((0, H - C), (0, 0)))
    w2rel = jnp.kron(eye8, w2p.T)
    w2rootp = jnp.pad(W2_root, ((0, H - C), (0, 0)))
    w2root = jnp.kron(eye8, w2rootp.T)
    b1t = jnp.tile(b1, 8)
    b2t = jnp.tile(jnp.pad(b2, (0, H - C)), 8)

    segsum = _sc_segsum()
    xrp, xrootp = _proj1(x, W1_rel, W1_root)
    parts1 = segsum(xrp.reshape(VROWS, 16), sd, zrows)
    hrp, hrootp = _mid(parts1.reshape(NC, PV, 128), xrootp, b1t, w2rel, w2root)
    parts2 = segsum(hrp.reshape(VROWS, 16), sd, zrows)
    return _outk(parts2.reshape(NC, PV, 128), hrootp, b2t)
